# Initial kernel scaffold; baseline (speedup 1.0000x reference)
import jax
import jax.numpy as jnp
from jax.experimental import pallas as pl


def kernel(sampled_textures, zbuf):
    raise NotImplementedError("write your pallas kernel here")


# speedup vs baseline: 10.9371x; 10.9371x over previous
import functools

import jax
import jax.numpy as jnp
from jax import lax
from jax.experimental import pallas as pl
from jax.experimental.pallas import tpu as pltpu
from jax.experimental.pallas import tpu_sc as plsc

B, H, W, K = 4, 384, 384, 8
P = B * H * W
NW = 32
PPW = P // NW
G = 1024
NG = G // 16
NITER = PPW // G

MAGIC = 2.0 ** 23


def _round_half_even(x):
    return jnp.where(x >= 0, (x + MAGIC) - MAGIC, (x - MAGIC) + MAGIC)


@functools.partial(
    pl.kernel,
    mesh=plsc.VectorSubcoreMesh(core_axis_name="c", subcore_axis_name="s"),
    compiler_params=pltpu.CompilerParams(needs_layout_passes=False),
    out_type=(
        jax.ShapeDtypeStruct((P * 4,), jnp.float32),
        jax.ShapeDtypeStruct((P,), jnp.float32),
        jax.ShapeDtypeStruct((P,), jnp.int32),
        jax.ShapeDtypeStruct((P * 32,), jnp.float32),
    ),
    scratch_types=[
        pltpu.VMEM((G * 40,), jnp.float32),
        pltpu.VMEM((G * 8,), jnp.float32),
        pltpu.VMEM((G * 4,), jnp.float32),
        pltpu.VMEM((G,), jnp.float32),
        pltpu.VMEM((G,), jnp.int32),
        pltpu.VMEM((G * 32,), jnp.float32),
    ],
)
def _sc_shader(st_h, z_h, comp_h, depth_h, lab_h, hum_h,
               st_v, z_v, comp_v, depth_v, lab_v, hum_v):
    wid = lax.axis_index("s") * 2 + lax.axis_index("c")
    base_px = wid * PPW
    lane = lax.iota(jnp.int32, 16)
    idx40 = lane * 40
    idx8 = lane * 8
    idx4 = lane * 4
    idx32 = lane * 32
    ones = jnp.full((16,), 1.0, jnp.float32)
    init_pat = jnp.where((lane & 3) == 3, 0.0, 1.0).astype(jnp.float32)

    def chunk(i, carry):
        off = base_px + i * G
        pltpu.sync_copy(st_h.at[pl.ds(off * 40, G * 40)], st_v)
        pltpu.sync_copy(z_h.at[pl.ds(off * 8, G * 8)], z_v)

        def group(g, c_in):
            pb = g * 16
            gidx = idx40 + g * (16 * 40)
            zidx = idx8 + g * (16 * 8)
            hbase = idx32 + g * 512
            for j in range(32):
                hum_v[pl.ds(g * 512 + j * 16, 16)] = init_pat
            rgb0 = ones
            rgb1 = ones
            rgb2 = ones
            aacc = jnp.zeros((16,), jnp.float32)
            depth = jnp.full((16,), 100.0, jnp.float32)
            labf = jnp.full((16,), float(K), jnp.float32)
            for k in range(K - 1, -1, -1):
                c0 = plsc.load_gather(st_v, [gidx + (k * 5 + 0)])
                c1 = plsc.load_gather(st_v, [gidx + (k * 5 + 1)])
                c2 = plsc.load_gather(st_v, [gidx + (k * 5 + 2)])
                a = plsc.load_gather(st_v, [gidx + (k * 5 + 3)])
                labk = plsc.load_gather(st_v, [gidx + (k * 5 + 4)])
                z = plsc.load_gather(z_v, [zidx + k])
                om = 1.0 - a
                rgb0 = c0 * a + rgb0 * om
                rgb1 = c1 * a + rgb1 * om
                rgb2 = c2 * a + rgb2 * om
                aacc = jnp.maximum(a, aacc)
                zvalid = z >= 0.0
                depth = jnp.where(z > 0.0, z * a + depth * om, depth)
                labf = jnp.where(zvalid & (a > 0.5), labk, labf)
                li = _round_half_even(labk).astype(jnp.int32)
                m = zvalid & (li >= 0) & (li < K)
                slot = hbase + li * 4
                plsc.store_scatter(hum_v, [slot], c0 * a + om, mask=m)
                plsc.store_scatter(hum_v, [slot + 1], c1 * a + om, mask=m)
                plsc.store_scatter(hum_v, [slot + 2], c2 * a + om, mask=m)
                plsc.store_scatter(hum_v, [slot + 3], a, mask=m)
            cbase = idx4 + g * 64
            plsc.store_scatter(comp_v, [cbase], rgb0)
            plsc.store_scatter(comp_v, [cbase + 1], rgb1)
            plsc.store_scatter(comp_v, [cbase + 2], rgb2)
            plsc.store_scatter(comp_v, [cbase + 3], aacc)
            depth_v[pl.ds(pb, 16)] = depth
            labf2 = jnp.where(labf > K - 0.5, jnp.float32(-1.0), labf)
            lab_v[pl.ds(pb, 16)] = _round_half_even(labf2).astype(jnp.int32)
            return c_in

        lax.fori_loop(0, NG, group, 0)
        pltpu.sync_copy(comp_v, comp_h.at[pl.ds(off * 4, G * 4)])
        pltpu.sync_copy(depth_v, depth_h.at[pl.ds(off, G)])
        pltpu.sync_copy(lab_v, lab_h.at[pl.ds(off, G)])
        pltpu.sync_copy(hum_v, hum_h.at[pl.ds(off * 32, G * 32)])
        return carry

    lax.fori_loop(0, NITER, chunk, 0)


def kernel(sampled_textures, zbuf):
    st_flat = sampled_textures.reshape(P * 40)
    z_flat = zbuf.reshape(P * 8)
    comp, depth, lab, hum = _sc_shader(st_flat, z_flat)
    composite_image = comp.reshape(B, H, W, 4)
    composite_depth = depth.reshape(B, H, W)
    composite_label = lab.reshape(B, H, W).astype(jnp.int64)
    human_images = hum.reshape(B, H, W, K, 4)
    return composite_image, composite_depth, composite_label, human_images


# speedup vs baseline: 110.8998x; 10.1398x over previous
import functools

import jax
import jax.numpy as jnp
from jax import lax
from jax.experimental import pallas as pl
from jax.experimental.pallas import tpu as pltpu
from jax.experimental.pallas import tpu_sc as plsc

B, H, W, K = 4, 384, 384, 8
P = B * H * W
NW = 32
PPW = P // NW
G = 768
NG = G // 16
NITER = PPW // G

MAGIC = 2.0 ** 23


def _round_half_even(x):
    return jnp.where(x >= 0, (x + MAGIC) - MAGIC, (x - MAGIC) + MAGIC)


@functools.partial(
    pl.kernel,
    mesh=plsc.VectorSubcoreMesh(core_axis_name="c", subcore_axis_name="s"),
    compiler_params=pltpu.CompilerParams(needs_layout_passes=False),
    out_type=(
        jax.ShapeDtypeStruct((4, P), jnp.float32),
        jax.ShapeDtypeStruct((P,), jnp.float32),
        jax.ShapeDtypeStruct((P,), jnp.int32),
        jax.ShapeDtypeStruct((K * 4, P), jnp.float32),
    ),
    scratch_types=[
        pltpu.VMEM((K * 5, G), jnp.float32),
        pltpu.VMEM((K, G), jnp.float32),
        pltpu.VMEM((4, G), jnp.float32),
        pltpu.VMEM((G,), jnp.float32),
        pltpu.VMEM((G,), jnp.int32),
        pltpu.VMEM((K * 4, G), jnp.float32),
    ],
)
def _sc_shader(st_h, z_h, comp_h, depth_h, lab_h, hum_h,
               st_v, z_v, comp_v, depth_v, lab_v, hum_v):
    wid = lax.axis_index("s") * 2 + lax.axis_index("c")
    lane = lax.iota(jnp.int32, 16)

    def chunk(i, carry):
        off = pl.multiple_of(wid * PPW + i * G, G)
        pltpu.sync_copy(st_h.at[:, pl.ds(off, G)], st_v)
        pltpu.sync_copy(z_h.at[:, pl.ds(off, G)], z_v)

        def group(g, c_in):
            pb = g * 16
            pv = lane + pb
            one = jnp.full((16,), 1.0, jnp.float32)
            for n in range(K):
                hum_v[n * 4 + 0, pl.ds(pb, 16)] = one
                hum_v[n * 4 + 1, pl.ds(pb, 16)] = one
                hum_v[n * 4 + 2, pl.ds(pb, 16)] = one
                hum_v[n * 4 + 3, pl.ds(pb, 16)] = one * 0.0
            rgb0 = one
            rgb1 = one
            rgb2 = one
            aacc = jnp.zeros((16,), jnp.float32)
            depth = jnp.full((16,), 100.0, jnp.float32)
            labf = jnp.full((16,), float(K), jnp.float32)
            for k in range(K - 1, -1, -1):
                c0 = st_v[k * 5 + 0, pl.ds(pb, 16)]
                c1 = st_v[k * 5 + 1, pl.ds(pb, 16)]
                c2 = st_v[k * 5 + 2, pl.ds(pb, 16)]
                a = st_v[k * 5 + 3, pl.ds(pb, 16)]
                labk = st_v[k * 5 + 4, pl.ds(pb, 16)]
                z = z_v[k, pl.ds(pb, 16)]
                om = 1.0 - a
                rgb0 = c0 * a + rgb0 * om
                rgb1 = c1 * a + rgb1 * om
                rgb2 = c2 * a + rgb2 * om
                aacc = jnp.maximum(a, aacc)
                zvalid = z >= 0.0
                depth = jnp.where(z > 0.0, z * a + depth * om, depth)
                labf = jnp.where(zvalid & (a > 0.5), labk, labf)
                li = _round_half_even(labk).astype(jnp.int32)
                m = zvalid & (li >= 0) & (li < K)
                li4 = li * 4
                plsc.store_scatter(hum_v, [li4, pv], c0 * a + om, mask=m)
                plsc.store_scatter(hum_v, [li4 + 1, pv], c1 * a + om, mask=m)
                plsc.store_scatter(hum_v, [li4 + 2, pv], c2 * a + om, mask=m)
                plsc.store_scatter(hum_v, [li4 + 3, pv], a, mask=m)
            comp_v[0, pl.ds(pb, 16)] = rgb0
            comp_v[1, pl.ds(pb, 16)] = rgb1
            comp_v[2, pl.ds(pb, 16)] = rgb2
            comp_v[3, pl.ds(pb, 16)] = aacc
            depth_v[pl.ds(pb, 16)] = depth
            labf2 = jnp.where(labf > K - 0.5, jnp.float32(-1.0), labf)
            lab_v[pl.ds(pb, 16)] = _round_half_even(labf2).astype(jnp.int32)
            return c_in

        lax.fori_loop(0, NG, group, 0)
        pltpu.sync_copy(comp_v, comp_h.at[:, pl.ds(off, G)])
        pltpu.sync_copy(depth_v, depth_h.at[pl.ds(off, G)])
        pltpu.sync_copy(lab_v, lab_h.at[pl.ds(off, G)])
        pltpu.sync_copy(hum_v, hum_h.at[:, pl.ds(off, G)])
        return carry

    lax.fori_loop(0, NITER, chunk, 0)


def kernel(sampled_textures, zbuf):
    st_t = jnp.transpose(sampled_textures, (3, 4, 0, 1, 2)).reshape(K * 5, P)
    z_t = jnp.transpose(zbuf, (3, 0, 1, 2)).reshape(K, P)
    comp_t, depth, lab, hum_t = _sc_shader(st_t, z_t)
    composite_image = jnp.transpose(comp_t.reshape(4, B, H, W), (1, 2, 3, 0))
    composite_depth = depth.reshape(B, H, W)
    composite_label = lab.reshape(B, H, W).astype(jnp.int64)
    human_images = jnp.transpose(hum_t.reshape(K, 4, B, H, W), (2, 3, 4, 0, 1))
    return composite_image, composite_depth, composite_label, human_images
